# baseline (speedup 1.0000x reference)
import functools

import jax
import jax.numpy as jnp
from jax import lax
from jax.experimental import pallas as pl
from jax.experimental.pallas import tpu as pltpu
from jax.experimental.pallas import tpu_sc as plsc

N = 50000
E = 500000
L = 100000
D = 128
NPAD = 50048
EPAD = 524288
EROWS = EPAD // 128
LPAD = 102400
LROWS = LPAD // 128
NC, NS = 2, 16
NW = NC * NS
INV_N = 1.0 / N
EPS = 1e-5


BN_ROWS = 1000


def _lin_stats_body(x_ref, wt_ref, b_ref, ssum_ref, ssq_ref):
    i = pl.program_id(0)
    y = jnp.dot(x_ref[...], wt_ref[...], preferred_element_type=jnp.float32)
    y = y + b_ref[...]
    s = jnp.sum(y, axis=0, keepdims=True)
    s2 = jnp.sum(y * y, axis=0, keepdims=True)

    @pl.when(i == 0)
    def _():
        ssum_ref[...] = jnp.zeros_like(ssum_ref)
        ssq_ref[...] = jnp.zeros_like(ssq_ref)

    ssum_ref[...] += s
    ssq_ref[...] += s2


def _lin_stats(x, wt, b):
    grid = (N // BN_ROWS,)
    return pl.pallas_call(
        _lin_stats_body,
        grid=grid,
        in_specs=[pl.BlockSpec((BN_ROWS, D), lambda i: (i, 0)),
                  pl.BlockSpec((D, D), lambda i: (0, 0)),
                  pl.BlockSpec((1, D), lambda i: (0, 0))],
        out_specs=[pl.BlockSpec((1, D), lambda i: (0, 0)),
                   pl.BlockSpec((1, D), lambda i: (0, 0))],
        out_shape=[jax.ShapeDtypeStruct((1, D), jnp.float32),
                   jax.ShapeDtypeStruct((1, D), jnp.float32)],
    )(x, wt, b)


def _bn_relu_2mm_body(has_enc, x_ref, wt_ref, b_ref, ssum_ref, ssq_ref,
                      g_ref, be_ref, wcat_ref, amsg_ref, aself_ref):
    if has_enc:
        y = jnp.dot(x_ref[...], wt_ref[...], preferred_element_type=jnp.float32)
        y = y + b_ref[...]
    else:
        y = x_ref[...]
    mu = ssum_ref[...] * INV_N
    var = ssq_ref[...] * INV_N - mu * mu
    rs = lax.rsqrt(var + EPS)
    h = jnp.maximum((y - mu) * (rs * g_ref[...]) + be_ref[...], 0.0)
    o = jnp.dot(h, wcat_ref[...], preferred_element_type=jnp.float32)
    amsg_ref[...] = o[:, 0:D]
    aself_ref[...] = o[:, D:2 * D]


def _bn_relu_2mm(x, wt, b, ssum, ssq, g, be, wcat_t, has_enc):
    grid = (N // BN_ROWS,)
    return pl.pallas_call(
        functools.partial(_bn_relu_2mm_body, has_enc),
        grid=grid,
        in_specs=[pl.BlockSpec((BN_ROWS, D), lambda i: (i, 0)),
                  pl.BlockSpec((D, D), lambda i: (0, 0)),
                  pl.BlockSpec((1, D), lambda i: (0, 0)),
                  pl.BlockSpec((1, D), lambda i: (0, 0)),
                  pl.BlockSpec((1, D), lambda i: (0, 0)),
                  pl.BlockSpec((1, D), lambda i: (0, 0)),
                  pl.BlockSpec((1, D), lambda i: (0, 0)),
                  pl.BlockSpec((D, 2 * D), lambda i: (0, 0))],
        out_specs=[pl.BlockSpec((BN_ROWS, D), lambda i: (i, 0)),
                   pl.BlockSpec((BN_ROWS, D), lambda i: (i, 0))],
        out_shape=[jax.ShapeDtypeStruct((N, D), jnp.float32),
                   jax.ShapeDtypeStruct((N, D), jnp.float32)],
    )(x, wt, b, ssum, ssq, g, be, wcat_t)


def _combine_stats_body(s2_ref, cnt_ref, aself_ref, bl_ref,
                        x_ref, ssum_ref, ssq_ref):
    i = pl.program_id(0)
    s = s2_ref[...]
    cnt = cnt_ref[0, :, 0:1]
    x = s / jnp.maximum(cnt, 1.0) + bl_ref[...] + aself_ref[...]
    x_ref[...] = x
    s1 = jnp.sum(x, axis=0, keepdims=True)
    sq = jnp.sum(x * x, axis=0, keepdims=True)

    @pl.when(i == 0)
    def _():
        ssum_ref[...] = jnp.zeros_like(ssum_ref)
        ssq_ref[...] = jnp.zeros_like(ssq_ref)

    ssum_ref[...] += s1
    ssq_ref[...] += sq


def _combine_stats(s2, cnt2, d, aself, bl):
    grid = (N // BN_ROWS,)
    return pl.pallas_call(
        _combine_stats_body,
        grid=grid,
        in_specs=[pl.BlockSpec((BN_ROWS, D), lambda i: (i, 0)),
                  pl.BlockSpec((1, BN_ROWS, D), lambda i, _d=d: (_d, i, 0)),
                  pl.BlockSpec((BN_ROWS, D), lambda i: (i, 0)),
                  pl.BlockSpec((1, D), lambda i: (0, 0))],
        out_specs=[pl.BlockSpec((BN_ROWS, D), lambda i: (i, 0)),
                   pl.BlockSpec((1, D), lambda i: (0, 0)),
                   pl.BlockSpec((1, D), lambda i: (0, 0))],
        out_shape=[jax.ShapeDtypeStruct((N, D), jnp.float32),
                   jax.ShapeDtypeStruct((1, D), jnp.float32),
                   jax.ShapeDtypeStruct((1, D), jnp.float32)],
    )(s2, cnt2, aself, bl)


def _decoder_body(gs_ref, gd_ref, su_ref, qu_ref, gu_ref, beu_ref,
                  sr_ref, qr_ref, gr_ref, ber_ref, o_ref):
    mu_u = su_ref[...] * INV_N
    au = lax.rsqrt(qu_ref[...] * INV_N - mu_u * mu_u + EPS) * gu_ref[...]
    cu = beu_ref[...] - mu_u * au
    mu_r = sr_ref[...] * INV_N
    ar = lax.rsqrt(qr_ref[...] * INV_N - mu_r * mu_r + EPS) * gr_ref[...]
    cr = ber_ref[...] - mu_r * ar
    zs = gs_ref[...] * au + cu
    zd = gd_ref[...] * ar + cr
    dot = jnp.sum(zs * zd, axis=1)
    ns = jnp.maximum(jnp.sqrt(jnp.sum(zs * zs, axis=1)), 1e-12)
    nd = jnp.maximum(jnp.sqrt(jnp.sum(zd * zd, axis=1)), 1e-12)
    o_ref[...] = (dot / (ns * nd)).reshape(1, 8, -1)


def _decoder(gs, gd, su, qu, gu, beu, sr, qr, gr, ber):
    bl = 2000
    grid = (L // bl,)
    vec = pl.BlockSpec((1, D), lambda i: (0, 0))
    out2 = pl.pallas_call(
        _decoder_body,
        grid=grid,
        in_specs=[pl.BlockSpec((bl, D), lambda i: (i, 0)),
                  pl.BlockSpec((bl, D), lambda i: (i, 0)),
                  vec, vec, vec, vec, vec, vec, vec, vec],
        out_specs=pl.BlockSpec((1, 8, bl // 8), lambda i: (i, 0, 0)),
        out_shape=jax.ShapeDtypeStruct((L // bl, 8, bl // 8), jnp.float32),
    )(gs, gd, su, qu, gu, beu, sr, qr, gr, ber)
    return out2.reshape(L)


def _mesh():
    return plsc.VectorSubcoreMesh(core_axis_name="c", subcore_axis_name="s")


RJ = 13696
RT = RJ // NS
NPASS = 2
SROWS_OUT = NPASS * NC * RJ
ESLABS = EPAD // 1024


def _ring(n_sets, lf, ds):
    lf(0, 0)
    m = (n_sets - 2) // 2
    if m > 0:
        def body(j, carry):
            lf(1, 2 * j + 1)
            ds(0, 2 * j)
            lf(0, 2 * j + 2)
            ds(1, 2 * j + 1)
            return carry
        lax.fori_loop(0, m, body, 0)
    k = 2 * m
    if n_sets - k == 2:
        lf(1, k + 1)
        ds(0, k)
        ds(1, k + 1)
    else:
        lf(1, k + 1)
        ds(0, k)
        lf(0, k + 2)
        ds(1, k + 1)
        ds(0, k + 2)


def _fill_rows(buf, value):
    def body(j, carry):
        for v in range(8):
            buf[j, pl.ds(16 * v, 16)] = jnp.full((16,), value, jnp.float32)
        return carry
    lax.fori_loop(0, 64, body, 0)


def _zero_acc_slice(acc, zsrc, s):
    for k in range(13):
        pltpu.sync_copy(zsrc, acc.at[pl.ds(s * RT + 64 * k, 64)])
    pltpu.sync_copy(zsrc.at[pl.ds(0, 24)], acc.at[pl.ds(s * RT + 832, 24)])


def _transform_dst(dstv, dstloc, base):
    def body(k, carry):
        r = k // 2
        off = (k % 2) * 64
        for v in range(4):
            d = dstv[r, pl.ds(off + 16 * v, 16)]
            l = d - base
            ok = (l >= 0) & (l < RJ)
            dstloc[k, pl.ds(16 * v, 16)] = jnp.where(ok, l, RJ)
        return carry
    lax.fori_loop(0, 16, body, 0)


def _sc_scatter_body(tbl, srcs, dsts, out, srcv, dstv, dstloc, rows, acc,
                     sem0, sem1):
    c = lax.axis_index("c")
    s = lax.axis_index("s")
    sems = (sem0, sem1)
    slabs_per_tile = ESLABS // NS
    for q in range(NPASS):
        base = q * (NC * RJ) + c * RJ
        _fill_rows(rows.at[0], 0.0)
        _zero_acc_slice(acc, rows.at[0], s)
        plsc.subcore_barrier()

        def slab_body(j, carry):
            slab = s * slabs_per_tile + j
            pltpu.sync_copy(srcs.at[slab], srcv)
            pltpu.sync_copy(dsts.at[slab], dstv)
            _transform_dst(dstv, dstloc, base)

            def lf(buf, k):
                idx = srcv.at[k // 2, pl.ds((k % 2) * 64, 64)]
                pltpu.make_async_copy(tbl.at[idx], rows.at[buf],
                                      sems[buf]).start()

            def ds(buf, k):
                idx = srcv.at[k // 2, pl.ds((k % 2) * 64, 64)]
                pltpu.make_async_copy(tbl.at[idx], rows.at[buf],
                                      sems[buf]).wait()
                pltpu.sync_copy(rows.at[buf], acc.at[dstloc.at[k]], add=True)

            _ring(16, lf, ds)
            return carry

        lax.fori_loop(0, slabs_per_tile, slab_body, 0)
        plsc.subcore_barrier()
        pltpu.sync_copy(acc.at[pl.ds(s * RT, RT)],
                        out.at[pl.ds(base + s * RT, RT)])
        plsc.subcore_barrier()


def _sc_counts_body(dsts2, out, dstv, dstloc, rows, acc, sem0, sem1):
    c = lax.axis_index("c")
    s = lax.axis_index("s")
    slabs_per_tile = ESLABS // NS
    _fill_rows(rows.at[0], 0.0)
    _fill_rows(rows.at[1], 1.0)
    for d in range(2):
        for q in range(NPASS):
            base = q * (NC * RJ) + c * RJ
            _zero_acc_slice(acc, rows.at[0], s)
            plsc.subcore_barrier()

            def slab_body(j, carry):
                slab = s * slabs_per_tile + j
                pltpu.sync_copy(dsts2.at[d, slab], dstv)
                _transform_dst(dstv, dstloc, base)

                def body(k, carry2):
                    pltpu.sync_copy(rows.at[1], acc.at[dstloc.at[k]],
                                    add=True)
                    return carry2
                lax.fori_loop(0, 16, body, 0)
                return carry

            lax.fori_loop(0, slabs_per_tile, slab_body, 0)
            plsc.subcore_barrier()
            pltpu.sync_copy(acc.at[pl.ds(s * RT, RT)],
                            out.at[d, pl.ds(base + s * RT, RT)])
            plsc.subcore_barrier()


def _sc_label_gather_body(zu, zr, lab2, gs, gd, idxv, rows, sem0, sem1):
    c = lax.axis_index("c")
    s = lax.axis_index("s")
    w = s * NC + c
    sems = (sem0, sem1)
    rows_per_tile = LROWS // NW
    row0 = w * rows_per_tile

    for d, (tbl, outref) in enumerate(((zu, gs), (zr, gd))):
        def lf(buf, st, _d=d, _tbl=tbl):
            base = row0 + st
            pltpu.sync_copy(lab2.at[_d, base], idxv.at[buf])
            pltpu.make_async_copy(_tbl.at[idxv.at[buf]], rows.at[buf],
                                  sems[buf]).start()

        def ds(buf, st, _tbl=tbl, _out=outref):
            pltpu.make_async_copy(_tbl.at[idxv.at[buf]], rows.at[buf],
                                  sems[buf]).wait()
            base = row0 + st
            pltpu.sync_copy(rows.at[buf], _out.at[pl.ds(base * 128, 128)])

        _ring(rows_per_tile, lf, ds)


@functools.cache
def _sc_kernels():
    mesh = _mesh()
    scatter = functools.partial(
        pl.kernel, mesh=mesh,
        out_type=jax.ShapeDtypeStruct((SROWS_OUT, D), jnp.float32),
        scratch_types=[
            pltpu.VMEM((8, 128), jnp.int32),
            pltpu.VMEM((8, 128), jnp.int32),
            pltpu.VMEM((16, 64), jnp.int32),
            pltpu.VMEM((2, 64, D), jnp.float32),
            pltpu.VMEM_SHARED((RJ + 8, D), jnp.float32),
            pltpu.SemaphoreType.DMA,
            pltpu.SemaphoreType.DMA,
        ],
    )(_sc_scatter_body)
    counts = functools.partial(
        pl.kernel, mesh=mesh,
        out_type=jax.ShapeDtypeStruct((2, SROWS_OUT, D), jnp.float32),
        scratch_types=[
            pltpu.VMEM((8, 128), jnp.int32),
            pltpu.VMEM((16, 64), jnp.int32),
            pltpu.VMEM((2, 64, D), jnp.float32),
            pltpu.VMEM_SHARED((RJ + 8, D), jnp.float32),
            pltpu.SemaphoreType.DMA,
            pltpu.SemaphoreType.DMA,
        ],
    )(_sc_counts_body)
    gather = functools.partial(
        pl.kernel, mesh=mesh,
        out_type=[jax.ShapeDtypeStruct((LPAD, D), jnp.float32),
                  jax.ShapeDtypeStruct((LPAD, D), jnp.float32)],
        scratch_types=[
            pltpu.VMEM((2, 128), jnp.int32),
            pltpu.VMEM((2, 128, D), jnp.float32),
            pltpu.SemaphoreType.DMA,
            pltpu.SemaphoreType.DMA,
        ],
    )(_sc_label_gather_body)
    return scatter, counts, gather


def _sc_scatter_sum(tbl, srcs, dsts):
    return _sc_kernels()[0](tbl, srcs, dsts)


def _sc_counts(dsts2):
    return _sc_kernels()[1](dsts2)


def _sc_label_gather(zu, zr, lab2):
    return _sc_kernels()[2](zu, zr, lab2)


def _prep_edges(edge_index):
    src = jnp.pad(edge_index[0], (0, EPAD - E))
    dst = jnp.pad(edge_index[1], (0, EPAD - E), constant_values=N)
    return src.reshape(ESLABS, 8, 128), dst.reshape(ESLABS, 8, 128)


def kernel(x_user, x_recipe, edge_index_u2r, edge_index_r2u, edge_label_index,
           W_user_lin, b_user_lin, W_recipe_lin, b_recipe_lin,
           g_u0, be_u0, g_r0, be_r0,
           c1_ur_Wl, c1_ur_bl, c1_ur_Wr, c1_ru_Wl, c1_ru_bl, c1_ru_Wr,
           g_u1, be_u1, g_r1, be_r1,
           c2_ur_Wl, c2_ur_bl, c2_ur_Wr, c2_ru_Wl, c2_ru_bl, c2_ru_Wr,
           g_u2, be_u2, g_r2, be_r2):
    row = lambda v: v.reshape(1, D)
    src_ur, dst_ur = _prep_edges(edge_index_u2r)
    src_ru, dst_ru = _prep_edges(edge_index_r2u)
    dsts2 = jnp.stack([dst_ur, dst_ru])
    lab2 = jnp.pad(edge_label_index, ((0, 0), (0, LPAD - L))).reshape(2, LROWS, 128)

    ssum_u, ssq_u = _lin_stats(x_user, W_user_lin.T, row(b_user_lin))
    ssum_r, ssq_r = _lin_stats(x_recipe, W_recipe_lin.T, row(b_recipe_lin))
    wcat_u1 = jnp.concatenate([c1_ur_Wl, c1_ru_Wr], axis=0).T
    wcat_r1 = jnp.concatenate([c1_ru_Wl, c1_ur_Wr], axis=0).T
    amsg_u, aself_u = _bn_relu_2mm(x_user, W_user_lin.T, row(b_user_lin),
                                   ssum_u, ssq_u, row(g_u0), row(be_u0),
                                   wcat_u1, True)
    amsg_r, aself_r = _bn_relu_2mm(x_recipe, W_recipe_lin.T, row(b_recipe_lin),
                                   ssum_r, ssq_r, row(g_r0), row(be_r0),
                                   wcat_r1, True)

    cnt2 = _sc_counts(dsts2)

    s2_r1 = _sc_scatter_sum(amsg_u, src_ur, dst_ur)
    s2_u1 = _sc_scatter_sum(amsg_r, src_ru, dst_ru)
    r1, ssum_r1, ssq_r1 = _combine_stats(s2_r1, cnt2, 0, aself_r, row(c1_ur_bl))
    u1, ssum_u1, ssq_u1 = _combine_stats(s2_u1, cnt2, 1, aself_u, row(c1_ru_bl))

    wcat_u2 = jnp.concatenate([c2_ur_Wl, c2_ru_Wr], axis=0).T
    wcat_r2 = jnp.concatenate([c2_ru_Wl, c2_ur_Wr], axis=0).T
    amsg2_u, aself2_u = _bn_relu_2mm(u1, W_user_lin.T, row(b_user_lin),
                                     ssum_u1, ssq_u1, row(g_u1), row(be_u1),
                                     wcat_u2, False)
    amsg2_r, aself2_r = _bn_relu_2mm(r1, W_recipe_lin.T, row(b_recipe_lin),
                                     ssum_r1, ssq_r1, row(g_r1), row(be_r1),
                                     wcat_r2, False)

    s2_zr = _sc_scatter_sum(amsg2_u, src_ur, dst_ur)
    s2_zu = _sc_scatter_sum(amsg2_r, src_ru, dst_ru)
    zr_raw, ssum_zr, ssq_zr = _combine_stats(s2_zr, cnt2, 0, aself2_r,
                                             row(c2_ur_bl))
    zu_raw, ssum_zu, ssq_zu = _combine_stats(s2_zu, cnt2, 1, aself2_u,
                                             row(c2_ru_bl))

    gs, gd = _sc_label_gather(zu_raw, zr_raw, lab2)
    return _decoder(gs, gd, ssum_zu, ssq_zu, row(g_u2), row(be_u2),
                    ssum_zr, ssq_zr, row(g_r2), row(be_r2))


# speedup vs baseline: 1.0005x; 1.0005x over previous
import functools

import jax
import jax.numpy as jnp
from jax import lax
from jax.experimental import pallas as pl
from jax.experimental.pallas import tpu as pltpu
from jax.experimental.pallas import tpu_sc as plsc

N = 50000
E = 500000
L = 100000
D = 128
NPAD = 50048
EPAD = 524288
LPAD = 102400
LROWS = LPAD // 128
NC, NS = 2, 16
NW = NC * NS
INV_N = 1.0 / N
EPS = 1e-5


BN_ROWS = 1000
CNTW = 16


def _lin_stats_body(x_ref, wt_ref, b_ref, ssum_ref, ssq_ref):
    i = pl.program_id(0)
    y = jnp.dot(x_ref[...], wt_ref[...], preferred_element_type=jnp.float32)
    y = y + b_ref[...]
    s = jnp.sum(y, axis=0, keepdims=True)
    s2 = jnp.sum(y * y, axis=0, keepdims=True)

    @pl.when(i == 0)
    def _():
        ssum_ref[...] = jnp.zeros_like(ssum_ref)
        ssq_ref[...] = jnp.zeros_like(ssq_ref)

    ssum_ref[...] += s
    ssq_ref[...] += s2


def _lin_stats(x, wt, b):
    grid = (N // BN_ROWS,)
    return pl.pallas_call(
        _lin_stats_body,
        grid=grid,
        in_specs=[pl.BlockSpec((BN_ROWS, D), lambda i: (i, 0)),
                  pl.BlockSpec((D, D), lambda i: (0, 0)),
                  pl.BlockSpec((1, D), lambda i: (0, 0))],
        out_specs=[pl.BlockSpec((1, D), lambda i: (0, 0)),
                   pl.BlockSpec((1, D), lambda i: (0, 0))],
        out_shape=[jax.ShapeDtypeStruct((1, D), jnp.float32),
                   jax.ShapeDtypeStruct((1, D), jnp.float32)],
    )(x, wt, b)


def _bn_relu_2mm_body(has_enc, x_ref, wt_ref, b_ref, ssum_ref, ssq_ref,
                      g_ref, be_ref, wcat_ref, amsg_ref, aself_ref):
    if has_enc:
        y = jnp.dot(x_ref[...], wt_ref[...], preferred_element_type=jnp.float32)
        y = y + b_ref[...]
    else:
        y = x_ref[...]
    mu = ssum_ref[...] * INV_N
    var = ssq_ref[...] * INV_N - mu * mu
    rs = lax.rsqrt(var + EPS)
    h = jnp.maximum((y - mu) * (rs * g_ref[...]) + be_ref[...], 0.0)
    o = jnp.dot(h, wcat_ref[...], preferred_element_type=jnp.float32)
    amsg_ref[...] = o[:, 0:D]
    aself_ref[...] = o[:, D:2 * D]


def _bn_relu_2mm(x, wt, b, ssum, ssq, g, be, wcat_t, has_enc):
    grid = (N // BN_ROWS,)
    return pl.pallas_call(
        functools.partial(_bn_relu_2mm_body, has_enc),
        grid=grid,
        in_specs=[pl.BlockSpec((BN_ROWS, D), lambda i: (i, 0)),
                  pl.BlockSpec((D, D), lambda i: (0, 0)),
                  pl.BlockSpec((1, D), lambda i: (0, 0)),
                  pl.BlockSpec((1, D), lambda i: (0, 0)),
                  pl.BlockSpec((1, D), lambda i: (0, 0)),
                  pl.BlockSpec((1, D), lambda i: (0, 0)),
                  pl.BlockSpec((1, D), lambda i: (0, 0)),
                  pl.BlockSpec((D, 2 * D), lambda i: (0, 0))],
        out_specs=[pl.BlockSpec((BN_ROWS, D), lambda i: (i, 0)),
                   pl.BlockSpec((BN_ROWS, D), lambda i: (i, 0))],
        out_shape=[jax.ShapeDtypeStruct((N, D), jnp.float32),
                   jax.ShapeDtypeStruct((N, D), jnp.float32)],
    )(x, wt, b, ssum, ssq, g, be, wcat_t)


def _combine_stats_body(s2_ref, cnt_ref, aself_ref, bl_ref,
                        x_ref, ssum_ref, ssq_ref):
    i = pl.program_id(0)
    s = s2_ref[...]
    cnt = cnt_ref[0, :, 0:1]
    x = s / jnp.maximum(cnt, 1.0) + bl_ref[...] + aself_ref[...]
    x_ref[...] = x
    s1 = jnp.sum(x, axis=0, keepdims=True)
    sq = jnp.sum(x * x, axis=0, keepdims=True)

    @pl.when(i == 0)
    def _():
        ssum_ref[...] = jnp.zeros_like(ssum_ref)
        ssq_ref[...] = jnp.zeros_like(ssq_ref)

    ssum_ref[...] += s1
    ssq_ref[...] += sq


def _combine_stats(s2, cnt2, d, aself, bl):
    grid = (N // BN_ROWS,)
    return pl.pallas_call(
        _combine_stats_body,
        grid=grid,
        in_specs=[pl.BlockSpec((BN_ROWS, D), lambda i: (i, 0)),
                  pl.BlockSpec((1, BN_ROWS, D), lambda i, _d=d: (_d, i, 0)),
                  pl.BlockSpec((BN_ROWS, D), lambda i: (i, 0)),
                  pl.BlockSpec((1, D), lambda i: (0, 0))],
        out_specs=[pl.BlockSpec((BN_ROWS, D), lambda i: (i, 0)),
                   pl.BlockSpec((1, D), lambda i: (0, 0)),
                   pl.BlockSpec((1, D), lambda i: (0, 0))],
        out_shape=[jax.ShapeDtypeStruct((N, D), jnp.float32),
                   jax.ShapeDtypeStruct((1, D), jnp.float32),
                   jax.ShapeDtypeStruct((1, D), jnp.float32)],
    )(s2, cnt2, aself, bl)


def _decoder_body(gs_ref, gd_ref, su_ref, qu_ref, gu_ref, beu_ref,
                  sr_ref, qr_ref, gr_ref, ber_ref, o_ref):
    mu_u = su_ref[...] * INV_N
    au = lax.rsqrt(qu_ref[...] * INV_N - mu_u * mu_u + EPS) * gu_ref[...]
    cu = beu_ref[...] - mu_u * au
    mu_r = sr_ref[...] * INV_N
    ar = lax.rsqrt(qr_ref[...] * INV_N - mu_r * mu_r + EPS) * gr_ref[...]
    cr = ber_ref[...] - mu_r * ar
    zs = gs_ref[...] * au + cu
    zd = gd_ref[...] * ar + cr
    dot = jnp.sum(zs * zd, axis=1)
    ns = jnp.maximum(jnp.sqrt(jnp.sum(zs * zs, axis=1)), 1e-12)
    nd = jnp.maximum(jnp.sqrt(jnp.sum(zd * zd, axis=1)), 1e-12)
    o_ref[...] = (dot / (ns * nd)).reshape(1, 8, -1)


def _decoder(gs, gd, su, qu, gu, beu, sr, qr, gr, ber):
    bl = 2000
    grid = (L // bl,)
    vec = pl.BlockSpec((1, D), lambda i: (0, 0))
    out2 = pl.pallas_call(
        _decoder_body,
        grid=grid,
        in_specs=[pl.BlockSpec((bl, D), lambda i: (i, 0)),
                  pl.BlockSpec((bl, D), lambda i: (i, 0)),
                  vec, vec, vec, vec, vec, vec, vec, vec],
        out_specs=pl.BlockSpec((1, 8, bl // 8), lambda i: (i, 0, 0)),
        out_shape=jax.ShapeDtypeStruct((L // bl, 8, bl // 8), jnp.float32),
    )(gs, gd, su, qu, gu, beu, sr, qr, gr, ber)
    return out2.reshape(L)


def _mesh():
    return plsc.VectorSubcoreMesh(core_axis_name="c", subcore_axis_name="s")


RJ = 13696
RT = RJ // NS
NPASS = 2
SROWS_OUT = NPASS * NC * RJ
RZ = NPAD // NS
ESLABS = EPAD // 1024


def _ring(n_sets, lf, ds):
    lf(0, 0)
    m = (n_sets - 2) // 2
    if m > 0:
        def body(j, carry):
            lf(1, 2 * j + 1)
            ds(0, 2 * j)
            lf(0, 2 * j + 2)
            ds(1, 2 * j + 1)
            return carry
        lax.fori_loop(0, m, body, 0)
    k = 2 * m
    if n_sets - k == 2:
        lf(1, k + 1)
        ds(0, k)
        ds(1, k + 1)
    else:
        lf(1, k + 1)
        ds(0, k)
        lf(0, k + 2)
        ds(1, k + 1)
        ds(0, k + 2)


def _fill_rows(buf, value, nrows, width):
    def body(j, carry):
        for v in range(width // 16):
            buf[j, pl.ds(16 * v, 16)] = jnp.full((16,), value, jnp.float32)
        return carry
    lax.fori_loop(0, nrows, body, 0)


def _zero_slice(acc, zsrc, row0, nrows):
    nfull, rem = nrows // 64, nrows % 64
    for k in range(nfull):
        pltpu.sync_copy(zsrc, acc.at[pl.ds(row0 + 64 * k, 64)])
    if rem:
        pltpu.sync_copy(zsrc.at[pl.ds(0, rem)],
                        acc.at[pl.ds(row0 + 64 * nfull, rem)])


def _transform_dst(dstv, dstloc, base):
    def body(k, carry):
        r = k // 2
        off = (k % 2) * 64
        for v in range(4):
            d = dstv[r, pl.ds(off + 16 * v, 16)]
            l = d - base
            ok = (l >= 0) & (l < RJ)
            dstloc[k, pl.ds(16 * v, 16)] = jnp.where(ok, l, RJ)
        return carry
    lax.fori_loop(0, 16, body, 0)


def _sc_scatter_body(tbl, srcs, dsts, out, srcv, dstv, dstloc, rows, acc,
                     sem0, sem1):
    c = lax.axis_index("c")
    s = lax.axis_index("s")
    sems = (sem0, sem1)
    slabs_per_tile = ESLABS // NS
    for q in range(NPASS):
        base = q * (NC * RJ) + c * RJ
        _fill_rows(rows.at[0], 0.0, 64, D)
        _zero_slice(acc, rows.at[0], s * RT, RT)
        plsc.subcore_barrier()

        def slab_body(j, carry):
            slab = s * slabs_per_tile + j
            pltpu.sync_copy(srcs.at[slab], srcv)
            pltpu.sync_copy(dsts.at[slab], dstv)
            _transform_dst(dstv, dstloc, base)

            def lf(buf, k):
                idx = srcv.at[k // 2, pl.ds((k % 2) * 64, 64)]
                pltpu.make_async_copy(tbl.at[idx], rows.at[buf],
                                      sems[buf]).start()

            def ds(buf, k):
                idx = srcv.at[k // 2, pl.ds((k % 2) * 64, 64)]
                pltpu.make_async_copy(tbl.at[idx], rows.at[buf],
                                      sems[buf]).wait()
                pltpu.sync_copy(rows.at[buf], acc.at[dstloc.at[k]], add=True)

            _ring(16, lf, ds)
            return carry

        lax.fori_loop(0, slabs_per_tile, slab_body, 0)
        plsc.subcore_barrier()
        pltpu.sync_copy(acc.at[pl.ds(s * RT, RT)],
                        out.at[pl.ds(base + s * RT, RT)])
        plsc.subcore_barrier()


def _sc_counts_body(dsts2, out, dstv, dstloc, rows, acc, sem0, sem1):
    c = lax.axis_index("c")
    s = lax.axis_index("s")
    slabs_per_tile = ESLABS // NS
    _fill_rows(rows.at[0], 0.0, 64, D)
    _fill_rows(rows.at[1], 1.0, 64, D)
    for d in range(2):
        for q in range(NPASS):
            base = q * (NC * RJ) + c * RJ
            _zero_slice(acc, rows.at[0], s * RT, RT)
            plsc.subcore_barrier()

            def slab_body(j, carry):
                slab = s * slabs_per_tile + j
                pltpu.sync_copy(dsts2.at[d, slab], dstv)
                _transform_dst(dstv, dstloc, base)

                def body(k, carry2):
                    pltpu.sync_copy(rows.at[1], acc.at[dstloc.at[k]],
                                    add=True)
                    return carry2
                lax.fori_loop(0, 16, body, 0)
                return carry

            lax.fori_loop(0, slabs_per_tile, slab_body, 0)
            plsc.subcore_barrier()
            pltpu.sync_copy(acc.at[pl.ds(s * RT, RT)],
                            out.at[d, pl.ds(base + s * RT, RT)])
            plsc.subcore_barrier()


def _sc_label_gather_body(zu, zr, lab2, gs, gd, idxv, rows, sem0, sem1):
    c = lax.axis_index("c")
    s = lax.axis_index("s")
    w = s * NC + c
    sems = (sem0, sem1)
    rows_per_tile = LROWS // NW
    row0 = w * rows_per_tile

    for d, (tbl, outref) in enumerate(((zu, gs), (zr, gd))):
        def lf(buf, st, _d=d, _tbl=tbl):
            base = row0 + st
            pltpu.sync_copy(lab2.at[_d, base], idxv.at[buf])
            pltpu.make_async_copy(_tbl.at[idxv.at[buf]], rows.at[buf],
                                  sems[buf]).start()

        def ds(buf, st, _tbl=tbl, _out=outref):
            pltpu.make_async_copy(_tbl.at[idxv.at[buf]], rows.at[buf],
                                  sems[buf]).wait()
            base = row0 + st
            pltpu.sync_copy(rows.at[buf], _out.at[pl.ds(base * 128, 128)])

        _ring(rows_per_tile, lf, ds)


@functools.cache
def _sc_kernels():
    mesh = _mesh()
    scatter = functools.partial(
        pl.kernel, mesh=mesh,
        out_type=jax.ShapeDtypeStruct((SROWS_OUT, D), jnp.float32),
        scratch_types=[
            pltpu.VMEM((8, 128), jnp.int32),
            pltpu.VMEM((8, 128), jnp.int32),
            pltpu.VMEM((16, 64), jnp.int32),
            pltpu.VMEM((2, 64, D), jnp.float32),
            pltpu.VMEM_SHARED((RJ + 8, D), jnp.float32),
            pltpu.SemaphoreType.DMA,
            pltpu.SemaphoreType.DMA,
        ],
    )(_sc_scatter_body)
    counts = functools.partial(
        pl.kernel, mesh=mesh,
        out_type=jax.ShapeDtypeStruct((2, SROWS_OUT, D), jnp.float32),
        scratch_types=[
            pltpu.VMEM((8, 128), jnp.int32),
            pltpu.VMEM((16, 64), jnp.int32),
            pltpu.VMEM((2, 64, D), jnp.float32),
            pltpu.VMEM_SHARED((RJ + 8, D), jnp.float32),
            pltpu.SemaphoreType.DMA,
            pltpu.SemaphoreType.DMA,
        ],
    )(_sc_counts_body)
    gather = functools.partial(
        pl.kernel, mesh=mesh,
        out_type=[jax.ShapeDtypeStruct((LPAD, D), jnp.float32),
                  jax.ShapeDtypeStruct((LPAD, D), jnp.float32)],
        scratch_types=[
            pltpu.VMEM((2, 128), jnp.int32),
            pltpu.VMEM((2, 128, D), jnp.float32),
            pltpu.SemaphoreType.DMA,
            pltpu.SemaphoreType.DMA,
        ],
    )(_sc_label_gather_body)
    return scatter, counts, gather


def _sc_scatter_sum(tbl, srcs, dsts):
    return _sc_kernels()[0](tbl, srcs, dsts)


def _sc_counts(dsts2):
    return _sc_kernels()[1](dsts2)


def _sc_label_gather(zu, zr, lab2):
    return _sc_kernels()[2](zu, zr, lab2)


def _prep_edges(edge_index):
    src = jnp.pad(edge_index[0], (0, EPAD - E))
    dst = jnp.pad(edge_index[1], (0, EPAD - E), constant_values=N)
    return src.reshape(ESLABS, 8, 128), dst.reshape(ESLABS, 8, 128)


def kernel(x_user, x_recipe, edge_index_u2r, edge_index_r2u, edge_label_index,
           W_user_lin, b_user_lin, W_recipe_lin, b_recipe_lin,
           g_u0, be_u0, g_r0, be_r0,
           c1_ur_Wl, c1_ur_bl, c1_ur_Wr, c1_ru_Wl, c1_ru_bl, c1_ru_Wr,
           g_u1, be_u1, g_r1, be_r1,
           c2_ur_Wl, c2_ur_bl, c2_ur_Wr, c2_ru_Wl, c2_ru_bl, c2_ru_Wr,
           g_u2, be_u2, g_r2, be_r2):
    row = lambda v: v.reshape(1, D)
    src_ur, dst_ur = _prep_edges(edge_index_u2r)
    src_ru, dst_ru = _prep_edges(edge_index_r2u)
    dsts2 = jnp.stack([dst_ur, dst_ru])
    lab2 = jnp.pad(edge_label_index, ((0, 0), (0, LPAD - L))).reshape(2, LROWS, 128)

    ssum_u, ssq_u = _lin_stats(x_user, W_user_lin.T, row(b_user_lin))
    ssum_r, ssq_r = _lin_stats(x_recipe, W_recipe_lin.T, row(b_recipe_lin))
    wcat_u1 = jnp.concatenate([c1_ur_Wl, c1_ru_Wr], axis=0).T
    wcat_r1 = jnp.concatenate([c1_ru_Wl, c1_ur_Wr], axis=0).T
    amsg_u, aself_u = _bn_relu_2mm(x_user, W_user_lin.T, row(b_user_lin),
                                   ssum_u, ssq_u, row(g_u0), row(be_u0),
                                   wcat_u1, True)
    amsg_r, aself_r = _bn_relu_2mm(x_recipe, W_recipe_lin.T, row(b_recipe_lin),
                                   ssum_r, ssq_r, row(g_r0), row(be_r0),
                                   wcat_r1, True)

    cnt2 = _sc_counts(dsts2)

    s2_r1 = _sc_scatter_sum(amsg_u, src_ur, dst_ur)
    s2_u1 = _sc_scatter_sum(amsg_r, src_ru, dst_ru)
    r1, ssum_r1, ssq_r1 = _combine_stats(s2_r1, cnt2, 0, aself_r, row(c1_ur_bl))
    u1, ssum_u1, ssq_u1 = _combine_stats(s2_u1, cnt2, 1, aself_u, row(c1_ru_bl))

    wcat_u2 = jnp.concatenate([c2_ur_Wl, c2_ru_Wr], axis=0).T
    wcat_r2 = jnp.concatenate([c2_ru_Wl, c2_ur_Wr], axis=0).T
    amsg2_u, aself2_u = _bn_relu_2mm(u1, W_user_lin.T, row(b_user_lin),
                                     ssum_u1, ssq_u1, row(g_u1), row(be_u1),
                                     wcat_u2, False)
    amsg2_r, aself2_r = _bn_relu_2mm(r1, W_recipe_lin.T, row(b_recipe_lin),
                                     ssum_r1, ssq_r1, row(g_r1), row(be_r1),
                                     wcat_r2, False)

    s2_zr = _sc_scatter_sum(amsg2_u, src_ur, dst_ur)
    s2_zu = _sc_scatter_sum(amsg2_r, src_ru, dst_ru)
    zr_raw, ssum_zr, ssq_zr = _combine_stats(s2_zr, cnt2, 0, aself2_r,
                                             row(c2_ur_bl))
    zu_raw, ssum_zu, ssq_zu = _combine_stats(s2_zu, cnt2, 1, aself2_u,
                                             row(c2_ru_bl))

    gs, gd = _sc_label_gather(zu_raw, zr_raw, lab2)
    return _decoder(gs, gd, ssum_zu, ssq_zu, row(g_u2), row(be_u2),
                    ssum_zr, ssq_zr, row(g_r2), row(be_r2))


# speedup vs baseline: 1.0023x; 1.0019x over previous
import functools

import jax
import jax.numpy as jnp
from jax import lax
from jax.experimental import pallas as pl
from jax.experimental.pallas import tpu as pltpu
from jax.experimental.pallas import tpu_sc as plsc

N = 50000
E = 500000
L = 100000
D = 128
NPAD = 50048
EPAD = 524288
LPAD = 102400
LROWS = LPAD // 128
NC, NS = 2, 16
NW = NC * NS
INV_N = 1.0 / N
EPS = 1e-5


BN_ROWS = 1000
CNTW = 16


def _lin_stats_body(x_ref, wt_ref, b_ref, ssum_ref, ssq_ref):
    i = pl.program_id(0)
    y = jnp.dot(x_ref[...], wt_ref[...], preferred_element_type=jnp.float32)
    y = y + b_ref[...]
    s = jnp.sum(y, axis=0, keepdims=True)
    s2 = jnp.sum(y * y, axis=0, keepdims=True)

    @pl.when(i == 0)
    def _():
        ssum_ref[...] = jnp.zeros_like(ssum_ref)
        ssq_ref[...] = jnp.zeros_like(ssq_ref)

    ssum_ref[...] += s
    ssq_ref[...] += s2


def _lin_stats(x, wt, b):
    grid = (N // BN_ROWS,)
    return pl.pallas_call(
        _lin_stats_body,
        grid=grid,
        in_specs=[pl.BlockSpec((BN_ROWS, D), lambda i: (i, 0)),
                  pl.BlockSpec((D, D), lambda i: (0, 0)),
                  pl.BlockSpec((1, D), lambda i: (0, 0))],
        out_specs=[pl.BlockSpec((1, D), lambda i: (0, 0)),
                   pl.BlockSpec((1, D), lambda i: (0, 0))],
        out_shape=[jax.ShapeDtypeStruct((1, D), jnp.float32),
                   jax.ShapeDtypeStruct((1, D), jnp.float32)],
    )(x, wt, b)


def _bn_relu_2mm_body(has_enc, x_ref, wt_ref, b_ref, ssum_ref, ssq_ref,
                      g_ref, be_ref, wcat_ref, amsg_ref, aself_ref):
    if has_enc:
        y = jnp.dot(x_ref[...], wt_ref[...], preferred_element_type=jnp.float32)
        y = y + b_ref[...]
    else:
        y = x_ref[...]
    mu = ssum_ref[...] * INV_N
    var = ssq_ref[...] * INV_N - mu * mu
    rs = lax.rsqrt(var + EPS)
    h = jnp.maximum((y - mu) * (rs * g_ref[...]) + be_ref[...], 0.0)
    o = jnp.dot(h, wcat_ref[...], preferred_element_type=jnp.float32)
    amsg_ref[...] = o[:, 0:D]
    aself_ref[...] = o[:, D:2 * D]


def _bn_relu_2mm(x, wt, b, ssum, ssq, g, be, wcat_t, has_enc):
    grid = (N // BN_ROWS,)
    return pl.pallas_call(
        functools.partial(_bn_relu_2mm_body, has_enc),
        grid=grid,
        in_specs=[pl.BlockSpec((BN_ROWS, D), lambda i: (i, 0)),
                  pl.BlockSpec((D, D), lambda i: (0, 0)),
                  pl.BlockSpec((1, D), lambda i: (0, 0)),
                  pl.BlockSpec((1, D), lambda i: (0, 0)),
                  pl.BlockSpec((1, D), lambda i: (0, 0)),
                  pl.BlockSpec((1, D), lambda i: (0, 0)),
                  pl.BlockSpec((1, D), lambda i: (0, 0)),
                  pl.BlockSpec((D, 2 * D), lambda i: (0, 0))],
        out_specs=[pl.BlockSpec((BN_ROWS, D), lambda i: (i, 0)),
                   pl.BlockSpec((BN_ROWS, D), lambda i: (i, 0))],
        out_shape=[jax.ShapeDtypeStruct((N, D), jnp.float32),
                   jax.ShapeDtypeStruct((N, D), jnp.float32)],
    )(x, wt, b, ssum, ssq, g, be, wcat_t)


def _combine_stats_body(s2_ref, cnt_ref, aself_ref, bl_ref,
                        x_ref, ssum_ref, ssq_ref):
    i = pl.program_id(0)
    s = s2_ref[...]
    cnt = cnt_ref[0, :, 0:1]
    x = s / jnp.maximum(cnt, 1.0) + bl_ref[...] + aself_ref[...]
    x_ref[...] = x
    s1 = jnp.sum(x, axis=0, keepdims=True)
    sq = jnp.sum(x * x, axis=0, keepdims=True)

    @pl.when(i == 0)
    def _():
        ssum_ref[...] = jnp.zeros_like(ssum_ref)
        ssq_ref[...] = jnp.zeros_like(ssq_ref)

    ssum_ref[...] += s1
    ssq_ref[...] += sq


def _combine_stats(s2, cnt2, d, aself, bl):
    grid = (N // BN_ROWS,)
    return pl.pallas_call(
        _combine_stats_body,
        grid=grid,
        in_specs=[pl.BlockSpec((BN_ROWS, D), lambda i: (i, 0)),
                  pl.BlockSpec((1, BN_ROWS, D), lambda i, _d=d: (_d, i, 0)),
                  pl.BlockSpec((BN_ROWS, D), lambda i: (i, 0)),
                  pl.BlockSpec((1, D), lambda i: (0, 0))],
        out_specs=[pl.BlockSpec((BN_ROWS, D), lambda i: (i, 0)),
                   pl.BlockSpec((1, D), lambda i: (0, 0)),
                   pl.BlockSpec((1, D), lambda i: (0, 0))],
        out_shape=[jax.ShapeDtypeStruct((N, D), jnp.float32),
                   jax.ShapeDtypeStruct((1, D), jnp.float32),
                   jax.ShapeDtypeStruct((1, D), jnp.float32)],
    )(s2, cnt2, aself, bl)


def _decoder_body(gs_ref, gd_ref, su_ref, qu_ref, gu_ref, beu_ref,
                  sr_ref, qr_ref, gr_ref, ber_ref, o_ref):
    mu_u = su_ref[...] * INV_N
    au = lax.rsqrt(qu_ref[...] * INV_N - mu_u * mu_u + EPS) * gu_ref[...]
    cu = beu_ref[...] - mu_u * au
    mu_r = sr_ref[...] * INV_N
    ar = lax.rsqrt(qr_ref[...] * INV_N - mu_r * mu_r + EPS) * gr_ref[...]
    cr = ber_ref[...] - mu_r * ar
    zs = gs_ref[...] * au + cu
    zd = gd_ref[...] * ar + cr
    dot = jnp.sum(zs * zd, axis=1)
    ns = jnp.maximum(jnp.sqrt(jnp.sum(zs * zs, axis=1)), 1e-12)
    nd = jnp.maximum(jnp.sqrt(jnp.sum(zd * zd, axis=1)), 1e-12)
    o_ref[...] = (dot / (ns * nd)).reshape(1, 8, -1)


def _decoder(gs, gd, su, qu, gu, beu, sr, qr, gr, ber):
    bl = 2000
    grid = (L // bl,)
    vec = pl.BlockSpec((1, D), lambda i: (0, 0))
    out2 = pl.pallas_call(
        _decoder_body,
        grid=grid,
        in_specs=[pl.BlockSpec((bl, D), lambda i: (i, 0)),
                  pl.BlockSpec((bl, D), lambda i: (i, 0)),
                  vec, vec, vec, vec, vec, vec, vec, vec],
        out_specs=pl.BlockSpec((1, 8, bl // 8), lambda i: (i, 0, 0)),
        out_shape=jax.ShapeDtypeStruct((L // bl, 8, bl // 8), jnp.float32),
    )(gs, gd, su, qu, gu, beu, sr, qr, gr, ber)
    return out2.reshape(L)


def _mesh():
    return plsc.VectorSubcoreMesh(core_axis_name="c", subcore_axis_name="s")


RJ = 12544
RT = RJ // NS
NPASS = 2
SROWS_OUT = NPASS * NC * RJ
RZ = NPAD // NS
ESLABS = EPAD // 1024


def _ring(n_sets, lf, ds):
    lf(0, 0)
    m = (n_sets - 2) // 2
    if m > 0:
        def body(j, carry):
            lf(1, 2 * j + 1)
            ds(0, 2 * j)
            lf(0, 2 * j + 2)
            ds(1, 2 * j + 1)
            return carry
        lax.fori_loop(0, m, body, 0)
    k = 2 * m
    if n_sets - k == 2:
        lf(1, k + 1)
        ds(0, k)
        ds(1, k + 1)
    else:
        lf(1, k + 1)
        ds(0, k)
        lf(0, k + 2)
        ds(1, k + 1)
        ds(0, k + 2)


def _fill_rows(buf, value, nrows, width):
    def body(j, carry):
        for v in range(width // 16):
            buf[j, pl.ds(16 * v, 16)] = jnp.full((16,), value, jnp.float32)
        return carry
    lax.fori_loop(0, nrows, body, 0)


def _zero_slice(acc, zsrc, row0, nrows):
    nfull, rem = nrows // 64, nrows % 64
    for k in range(nfull):
        pltpu.sync_copy(zsrc, acc.at[pl.ds(row0 + 64 * k, 64)])
    if rem:
        pltpu.sync_copy(zsrc.at[pl.ds(0, rem)],
                        acc.at[pl.ds(row0 + 64 * nfull, rem)])


def _transform_dst(dstv, dstloc, base):
    def body(k, carry):
        r = k // 2
        off = (k % 2) * 64
        for v in range(4):
            d = dstv[r, pl.ds(off + 16 * v, 16)]
            l = d - base
            ok = (l >= 0) & (l < RJ)
            dstloc[k, pl.ds(16 * v, 16)] = jnp.where(ok, l, RJ)
        return carry
    lax.fori_loop(0, 16, body, 0)


def _sc_scatter_body(tbl, srcs, dsts, out, srcv, dstv, dstloc, rows, acc,
                     g0, g1, g2, s0, s1, s2):
    c = lax.axis_index("c")
    s = lax.axis_index("s")
    gsem = (g0, g1, g2)
    ssem = (s0, s1, s2)
    slabs_per_tile = ESLABS // NS
    for q in range(NPASS):
        base = q * (NC * RJ) + c * RJ
        _fill_rows(rows.at[0], 0.0, 64, D)
        _zero_slice(acc, rows.at[0], s * RT, RT)
        plsc.subcore_barrier()

        def slab_body(j, carry):
            slab = s * slabs_per_tile + j
            pltpu.sync_copy(srcs.at[slab], srcv)
            pltpu.sync_copy(dsts.at[slab], dstv)
            _transform_dst(dstv, dstloc, base)

            def gd(k):
                b = k % 3
                idx = srcv.at[k // 2, pl.ds((k % 2) * 64, 64)]
                return pltpu.make_async_copy(tbl.at[idx], rows.at[b],
                                             gsem[b])

            scat = {}
            gd(0).start()
            gd(1).start()
            for k in range(16):
                b = k % 3
                gd(k).wait()
                scat[k] = pltpu.async_copy(rows.at[b], acc.at[dstloc.at[k]],
                                           ssem[b], add=True)
                nk = k + 2
                if nk < 16:
                    if nk >= 3:
                        scat[nk - 3].wait()
                    gd(nk).start()
            for k in range(13, 16):
                scat[k].wait()
            return carry

        lax.fori_loop(0, slabs_per_tile, slab_body, 0)
        plsc.subcore_barrier()
        pltpu.sync_copy(acc.at[pl.ds(s * RT, RT)],
                        out.at[pl.ds(base + s * RT, RT)])
        plsc.subcore_barrier()


def _sc_counts_body(dsts2, out, dstv, dstloc, rows, acc, s0, s1, s2, s3):
    c = lax.axis_index("c")
    s = lax.axis_index("s")
    ssem = (s0, s1, s2, s3)
    slabs_per_tile = ESLABS // NS
    _fill_rows(rows.at[0], 0.0, 64, D)
    _fill_rows(rows.at[1], 1.0, 64, D)
    for d in range(2):
        for q in range(NPASS):
            base = q * (NC * RJ) + c * RJ
            _zero_slice(acc, rows.at[0], s * RT, RT)
            plsc.subcore_barrier()

            def slab_body(j, carry):
                slab = s * slabs_per_tile + j
                pltpu.sync_copy(dsts2.at[d, slab], dstv)
                _transform_dst(dstv, dstloc, base)

                scat = {}
                for k in range(16):
                    if k >= 4:
                        scat[k - 4].wait()
                    scat[k] = pltpu.async_copy(rows.at[1],
                                               acc.at[dstloc.at[k]],
                                               ssem[k % 4], add=True)
                for k in range(12, 16):
                    scat[k].wait()
                return carry

            lax.fori_loop(0, slabs_per_tile, slab_body, 0)
            plsc.subcore_barrier()
            pltpu.sync_copy(acc.at[pl.ds(s * RT, RT)],
                            out.at[d, pl.ds(base + s * RT, RT)])
            plsc.subcore_barrier()


def _sc_label_gather_body(zu, zr, lab2, gs, gd, idxv, rows, sem0, sem1):
    c = lax.axis_index("c")
    s = lax.axis_index("s")
    w = s * NC + c
    sems = (sem0, sem1)
    rows_per_tile = LROWS // NW
    row0 = w * rows_per_tile

    for d, (tbl, outref) in enumerate(((zu, gs), (zr, gd))):
        def lf(buf, st, _d=d, _tbl=tbl):
            base = row0 + st
            pltpu.sync_copy(lab2.at[_d, base], idxv.at[buf])
            pltpu.make_async_copy(_tbl.at[idxv.at[buf]], rows.at[buf],
                                  sems[buf]).start()

        def ds(buf, st, _tbl=tbl, _out=outref):
            pltpu.make_async_copy(_tbl.at[idxv.at[buf]], rows.at[buf],
                                  sems[buf]).wait()
            base = row0 + st
            pltpu.sync_copy(rows.at[buf], _out.at[pl.ds(base * 128, 128)])

        _ring(rows_per_tile, lf, ds)


@functools.cache
def _sc_kernels():
    mesh = _mesh()
    scatter = functools.partial(
        pl.kernel, mesh=mesh,
        out_type=jax.ShapeDtypeStruct((SROWS_OUT, D), jnp.float32),
        scratch_types=[
            pltpu.VMEM((8, 128), jnp.int32),
            pltpu.VMEM((8, 128), jnp.int32),
            pltpu.VMEM((16, 64), jnp.int32),
            pltpu.VMEM((3, 64, D), jnp.float32),
            pltpu.VMEM_SHARED((RJ + 8, D), jnp.float32),
            pltpu.SemaphoreType.DMA,
            pltpu.SemaphoreType.DMA,
            pltpu.SemaphoreType.DMA,
            pltpu.SemaphoreType.DMA,
            pltpu.SemaphoreType.DMA,
            pltpu.SemaphoreType.DMA,
        ],
    )(_sc_scatter_body)
    counts = functools.partial(
        pl.kernel, mesh=mesh,
        out_type=jax.ShapeDtypeStruct((2, SROWS_OUT, D), jnp.float32),
        scratch_types=[
            pltpu.VMEM((8, 128), jnp.int32),
            pltpu.VMEM((16, 64), jnp.int32),
            pltpu.VMEM((2, 64, D), jnp.float32),
            pltpu.VMEM_SHARED((RJ + 8, D), jnp.float32),
            pltpu.SemaphoreType.DMA,
            pltpu.SemaphoreType.DMA,
            pltpu.SemaphoreType.DMA,
            pltpu.SemaphoreType.DMA,
        ],
    )(_sc_counts_body)
    gather = functools.partial(
        pl.kernel, mesh=mesh,
        out_type=[jax.ShapeDtypeStruct((LPAD, D), jnp.float32),
                  jax.ShapeDtypeStruct((LPAD, D), jnp.float32)],
        scratch_types=[
            pltpu.VMEM((2, 128), jnp.int32),
            pltpu.VMEM((2, 128, D), jnp.float32),
            pltpu.SemaphoreType.DMA,
            pltpu.SemaphoreType.DMA,
        ],
    )(_sc_label_gather_body)
    return scatter, counts, gather


def _sc_scatter_sum(tbl, srcs, dsts):
    return _sc_kernels()[0](tbl, srcs, dsts)


def _sc_counts(dsts2):
    return _sc_kernels()[1](dsts2)


def _sc_label_gather(zu, zr, lab2):
    return _sc_kernels()[2](zu, zr, lab2)


def _prep_edges(edge_index):
    src = jnp.pad(edge_index[0], (0, EPAD - E))
    dst = jnp.pad(edge_index[1], (0, EPAD - E), constant_values=N)
    return src.reshape(ESLABS, 8, 128), dst.reshape(ESLABS, 8, 128)


def kernel(x_user, x_recipe, edge_index_u2r, edge_index_r2u, edge_label_index,
           W_user_lin, b_user_lin, W_recipe_lin, b_recipe_lin,
           g_u0, be_u0, g_r0, be_r0,
           c1_ur_Wl, c1_ur_bl, c1_ur_Wr, c1_ru_Wl, c1_ru_bl, c1_ru_Wr,
           g_u1, be_u1, g_r1, be_r1,
           c2_ur_Wl, c2_ur_bl, c2_ur_Wr, c2_ru_Wl, c2_ru_bl, c2_ru_Wr,
           g_u2, be_u2, g_r2, be_r2):
    row = lambda v: v.reshape(1, D)
    src_ur, dst_ur = _prep_edges(edge_index_u2r)
    src_ru, dst_ru = _prep_edges(edge_index_r2u)
    dsts2 = jnp.stack([dst_ur, dst_ru])
    lab2 = jnp.pad(edge_label_index, ((0, 0), (0, LPAD - L))).reshape(2, LROWS, 128)

    ssum_u, ssq_u = _lin_stats(x_user, W_user_lin.T, row(b_user_lin))
    ssum_r, ssq_r = _lin_stats(x_recipe, W_recipe_lin.T, row(b_recipe_lin))
    wcat_u1 = jnp.concatenate([c1_ur_Wl, c1_ru_Wr], axis=0).T
    wcat_r1 = jnp.concatenate([c1_ru_Wl, c1_ur_Wr], axis=0).T
    amsg_u, aself_u = _bn_relu_2mm(x_user, W_user_lin.T, row(b_user_lin),
                                   ssum_u, ssq_u, row(g_u0), row(be_u0),
                                   wcat_u1, True)
    amsg_r, aself_r = _bn_relu_2mm(x_recipe, W_recipe_lin.T, row(b_recipe_lin),
                                   ssum_r, ssq_r, row(g_r0), row(be_r0),
                                   wcat_r1, True)

    cnt2 = _sc_counts(dsts2)

    s2_r1 = _sc_scatter_sum(amsg_u, src_ur, dst_ur)
    s2_u1 = _sc_scatter_sum(amsg_r, src_ru, dst_ru)
    r1, ssum_r1, ssq_r1 = _combine_stats(s2_r1, cnt2, 0, aself_r, row(c1_ur_bl))
    u1, ssum_u1, ssq_u1 = _combine_stats(s2_u1, cnt2, 1, aself_u, row(c1_ru_bl))

    wcat_u2 = jnp.concatenate([c2_ur_Wl, c2_ru_Wr], axis=0).T
    wcat_r2 = jnp.concatenate([c2_ru_Wl, c2_ur_Wr], axis=0).T
    amsg2_u, aself2_u = _bn_relu_2mm(u1, W_user_lin.T, row(b_user_lin),
                                     ssum_u1, ssq_u1, row(g_u1), row(be_u1),
                                     wcat_u2, False)
    amsg2_r, aself2_r = _bn_relu_2mm(r1, W_recipe_lin.T, row(b_recipe_lin),
                                     ssum_r1, ssq_r1, row(g_r1), row(be_r1),
                                     wcat_r2, False)

    s2_zr = _sc_scatter_sum(amsg2_u, src_ur, dst_ur)
    s2_zu = _sc_scatter_sum(amsg2_r, src_ru, dst_ru)
    zr_raw, ssum_zr, ssq_zr = _combine_stats(s2_zr, cnt2, 0, aself2_r,
                                             row(c2_ur_bl))
    zu_raw, ssum_zu, ssq_zu = _combine_stats(s2_zu, cnt2, 1, aself2_u,
                                             row(c2_ru_bl))

    gs, gd = _sc_label_gather(zu_raw, zr_raw, lab2)
    return _decoder(gs, gd, ssum_zu, ssq_zu, row(g_u2), row(be_u2),
                    ssum_zr, ssq_zr, row(g_r2), row(be_r2))
